# baseline (speedup 1.0000x reference)
import jax
import jax.numpy as jnp
from jax import lax
from jax.experimental import pallas as pl
from jax.experimental.pallas import tpu as pltpu
from jax.experimental.pallas import tpu_sc as plsc

_LAMBDA_POINT = 20.0

_B, _T, _H, _W = 8, 16, 256, 256
_N = 512
_BT = _B * _T
_NC, _NS, _L = 2, 16, 16
_NW = _NC * _NS
_BT_PER_W = _BT // _NW
_PTS_PER_W = _BT_PER_W * _N
_HH = _H // 2
_NHALF = _BT_PER_W * 2


def _point_loss_sc(pred_hbm, rows_hbm, cols_hbm, sv_hbm, out_hbm,
                   rows_v, cols_v, buf0, buf1, sv_v, acc_v, sem0, sem1):
    cid = lax.axis_index("c")
    sid = lax.axis_index("s")
    wid = cid * _NS + sid

    pltpu.sync_copy(rows_hbm, rows_v)
    pltpu.sync_copy(cols_hbm, cols_v)
    pltpu.sync_copy(sv_hbm.at[pl.ds(wid * _PTS_PER_W, _PTS_PER_W)], sv_v)

    bufs = (buf0, buf1)
    sems = (sem0, sem1)
    base_row = wid * _BT_PER_W * _H

    def start(h):
        return pltpu.async_copy(
            pred_hbm.at[pl.ds(base_row + h * _HH, _HH), :],
            bufs[h % 2], sems[h % 2])

    pending = start(0)
    acc = jnp.zeros((_L,), jnp.float32)
    for h in range(_NHALF):
        nxt = start(h + 1) if h + 1 < _NHALF else None
        pending.wait()
        pending = nxt
        img, half = h // 2, h % 2
        buf = bufs[h % 2]

        def chunk(i, a, img=img, half=half, buf=buf):
            sl = pl.ds(i * _L, _L)
            r = rows_v[sl]
            c = cols_v[sl]
            mask = lax.shift_right_logical(r, 7) == half
            g = plsc.load_gather(buf, [lax.bitwise_and(r, _HH - 1), c], mask=mask)
            d = g - sv_v[pl.ds(img * _N + i * _L, _L)]
            return a + jnp.where(mask, d * d, 0.0)

        acc = lax.fori_loop(0, _N // _L, chunk, acc)

    acc_v[...] = acc * (_LAMBDA_POINT / (_BT * _N))
    pltpu.sync_copy(acc_v, out_hbm.at[wid])


@jax.jit
def kernel(pred, s_coords, s_values):
    pred2d = pred.reshape(_BT * _H, _W)
    rows = s_coords[:, 0].astype(jnp.int32)
    cols = s_coords[:, 1].astype(jnp.int32)
    sv = s_values.reshape(-1).astype(jnp.float32)

    mesh = plsc.VectorSubcoreMesh(core_axis_name="c", subcore_axis_name="s")
    f = pl.kernel(
        _point_loss_sc,
        mesh=mesh,
        out_type=jax.ShapeDtypeStruct((_NW, _L), jnp.float32),
        compiler_params=pltpu.CompilerParams(needs_layout_passes=False),
        scratch_types=[
            pltpu.VMEM((_N,), jnp.int32),
            pltpu.VMEM((_N,), jnp.int32),
            pltpu.VMEM((_HH, _W), jnp.float32),
            pltpu.VMEM((_HH, _W), jnp.float32),
            pltpu.VMEM((_PTS_PER_W,), jnp.float32),
            pltpu.VMEM((_L,), jnp.float32),
            pltpu.SemaphoreType.DMA,
            pltpu.SemaphoreType.DMA,
        ],
    )
    partial = f(pred2d, rows, cols, sv)
    return jnp.sum(partial)


# speedup vs baseline: 1.0238x; 1.0238x over previous
import jax
import jax.numpy as jnp
from jax import lax
from jax.experimental import pallas as pl
from jax.experimental.pallas import tpu as pltpu
from jax.experimental.pallas import tpu_sc as plsc

_LAMBDA_POINT = 20.0

_B, _T, _H, _W = 8, 16, 256, 256
_N = 512
_BT = _B * _T
_NC, _NS, _L = 2, 16, 16
_NW = _NC * _NS
_BT_PER_W = _BT // _NW
_PTS_PER_W = _BT_PER_W * _N


def _point_loss_sc(pred_hbm, coords_hbm, sv_hbm, out_hbm,
                   coords_v, rows_v, cols_v, img_v, sv_v, acc_v, sem):
    cid = lax.axis_index("c")
    sid = lax.axis_index("s")
    wid = cid * _NS + sid

    pltpu.sync_copy(coords_hbm, coords_v)
    pltpu.sync_copy(sv_hbm.at[pl.ds(wid * _PTS_PER_W, _PTS_PER_W)], sv_v)

    def deint(i, carry):
        sl = pl.ds(i * _L, _L)
        pos = (2 * _L) * i + 2 * lax.iota(jnp.int32, _L)
        rows_v[sl] = plsc.load_gather(coords_v, [pos])
        cols_v[sl] = plsc.load_gather(coords_v, [pos + 1])
        return carry

    lax.fori_loop(0, _N // _L, deint, 0)

    acc = jnp.zeros((_L,), jnp.float32)
    for img in range(_BT_PER_W):
        row0 = (wid * _BT_PER_W + img) * _H
        pltpu.async_copy(pred_hbm.at[pl.ds(row0, _H), :], img_v, sem).wait()

        def chunk(i, a, img=img):
            sl = pl.ds(i * _L, _L)
            g = plsc.load_gather(img_v, [rows_v[sl], cols_v[sl]])
            d = g - sv_v[pl.ds(img * _N + i * _L, _L)]
            return a + d * d

        acc = lax.fori_loop(0, _N // _L, chunk, acc)

    acc_v[...] = acc * (_LAMBDA_POINT / (_BT * _N))
    pltpu.sync_copy(acc_v, out_hbm.at[wid])


@jax.jit
def kernel(pred, s_coords, s_values):
    pred2d = pred.reshape(_BT * _H, _W)
    coords = s_coords.astype(jnp.int32).reshape(-1)
    sv = s_values.reshape(-1).astype(jnp.float32)

    mesh = plsc.VectorSubcoreMesh(core_axis_name="c", subcore_axis_name="s")
    f = pl.kernel(
        _point_loss_sc,
        mesh=mesh,
        out_type=jax.ShapeDtypeStruct((_NW, _L), jnp.float32),
        compiler_params=pltpu.CompilerParams(needs_layout_passes=False),
        scratch_types=[
            pltpu.VMEM((2 * _N,), jnp.int32),
            pltpu.VMEM((_N,), jnp.int32),
            pltpu.VMEM((_N,), jnp.int32),
            pltpu.VMEM((_H, _W), jnp.float32),
            pltpu.VMEM((_PTS_PER_W,), jnp.float32),
            pltpu.VMEM((_L,), jnp.float32),
            pltpu.SemaphoreType.DMA,
        ],
    )
    partial = f(pred2d, coords, sv)
    return jnp.sum(partial)
